# Initial kernel scaffold; baseline (speedup 1.0000x reference)
import jax
import jax.numpy as jnp
from jax.experimental import pallas as pl


def kernel(hx, current_location, y_path, image_data, Wc, bc, Wv, bv, Wscf, bscf, Wih, Whh, bih, bhh, Wdy, bdy, Wsc, bsc):
    raise NotImplementedError("write your pallas kernel here")


# speedup vs baseline: 1.1294x; 1.1294x over previous
import math
import jax
import jax.numpy as jnp
from jax.experimental import pallas as pl
from jax.experimental.pallas import tpu as pltpu

K = 4
BN = 8
HZ = 10.0
SPH, SPW = 6, 6
NB = SPH * SPW
R0, R1 = 0.5, 4.0
RSTEP = (R1 - R0) / SPH
TSTEP = 2.0 * math.pi / SPW
SEQ = 40
HM = 80
R = K * BN


def _body(hx_ref, fm_ref, lin_ref, A_ref, yfv_ref, Wscf_ref,
          Wr_ref, Wz_ref, Wn_ref, Ur_ref, Uz_ref, Un_ref,
          bir_ref, biz_ref, bin_ref, bhr_ref, bhz_ref, bhn_ref,
          Wdy_ref, bdy_ref, Wsc_ref, bsc_ref,
          dy_ref, sc_ref):
    hx0 = hx_ref[...]
    hx_init = jnp.concatenate([hx0, hx0, hx0, hx0], axis=0)

    def step(i, carry):
        hx, hsum = carry
        hidden = hx[0:BN]

        rhalf = jnp.zeros((R, 48), jnp.float32)
        for b in range(NB):
            A_b = A_ref[pl.ds(i * (NB * R) + b * R, R), :]
            sp_b = jnp.dot(A_b, hidden,
                           preferred_element_type=jnp.float32)
            rhalf = rhalf + jnp.dot(sp_b, Wscf_ref[b * 48:(b + 1) * 48, :],
                                    preferred_element_type=jnp.float32)

        rows = [fm_ref[pl.ds(lin_ref[i, r], 1), :] for r in range(R)]
        feat = jnp.concatenate(rows, axis=0)

        yfv_i = yfv_ref[pl.ds(i * R, R), :]
        x = jnp.concatenate([feat, yfv_i, rhalf], axis=1)

        r_g = jax.nn.sigmoid(
            jnp.dot(x, Wr_ref[...], preferred_element_type=jnp.float32)
            + bir_ref[...]
            + jnp.dot(hx, Ur_ref[...], preferred_element_type=jnp.float32)
            + bhr_ref[...])
        z_g = jax.nn.sigmoid(
            jnp.dot(x, Wz_ref[...], preferred_element_type=jnp.float32)
            + biz_ref[...]
            + jnp.dot(hx, Uz_ref[...], preferred_element_type=jnp.float32)
            + bhz_ref[...])
        gh_n = jnp.dot(hx, Un_ref[...],
                       preferred_element_type=jnp.float32) + bhn_ref[...]
        gi_n = jnp.dot(x, Wn_ref[...],
                       preferred_element_type=jnp.float32) + bin_ref[...]
        n_g = jnp.tanh(gi_n + r_g * gh_n)
        hx_new = (1.0 - z_g) * n_g + z_g * hx
        return hx_new, hsum + hx_new

    hx_fin, hsum = jax.lax.fori_loop(
        0, SEQ, step, (hx_init, jnp.zeros((R, 48), jnp.float32)))

    dy_ref[...] = jnp.dot(hx_fin, Wdy_ref[...],
                          preferred_element_type=jnp.float32) + bdy_ref[...]
    sc_ref[...] = jnp.dot(hsum, Wsc_ref[...],
                          preferred_element_type=jnp.float32) \
        + float(SEQ) * bsc_ref[...]


def bscf_bcast(Wscf_ref, bir_ref):
    return 0.0


def kernel(hx, current_location, y_path, image_data, Wc, bc, Wv, bv,
           Wscf, bscf, Wih, Whh, bih, bhh, Wdy, bdy, Wsc, bsc):
    fm = jax.lax.conv_general_dilated(
        image_data, Wc, (2, 2), "SAME",
        dimension_numbers=("NCHW", "OIHW", "NCHW"))
    fm = jax.nn.relu(fm + bc[None, :, None, None])
    fm_flat = fm[0].reshape(32, HM * HM).T

    yp_t = jnp.swapaxes(y_path, 0, 1)

    ua = HM // 2 - yp_t[..., 1].astype(jnp.int32)
    va = HM // 2 - yp_t[..., 0].astype(jnp.int32)
    lin = jnp.clip(ua, 0, HM - 1) * HM + jnp.clip(va, 0, HM - 1)
    lin_all = lin.reshape(SEQ, R).astype(jnp.int32)

    a = yp_t[:, :, :, None, :]
    b = yp_t[:, :, None, :, :]
    c = b - a
    dist = jnp.linalg.norm(c, axis=-1)
    mask = (dist <= R1) & (dist >= R0)
    dd = jnp.where(dist < 1e-10, 1e-10, dist)
    costh = jnp.clip(c[..., 0] / dd, -1.0, 1.0)
    theta = jnp.where(c[..., 1] < 0,
                      2.0 * math.pi - jnp.arccos(costh),
                      jnp.arccos(costh))
    ub = ((dist - R0) / RSTEP).astype(jnp.int32)
    vb = (theta / TSTEP).astype(jnp.int32)
    idx = jnp.where(mask, ub * SPW + vb, NB)
    tj = (jax.lax.broadcasted_iota(jnp.int32, (BN, BN), 0)
          == jax.lax.broadcasted_iota(jnp.int32, (BN, BN), 1))
    idx = jnp.where(tj[None, None], NB, idx)

    oh = (idx[..., None] == jnp.arange(NB)).astype(jnp.float32)
    later = jnp.cumsum(oh[:, :, :, ::-1], axis=3)[:, :, :, ::-1] - oh
    w = oh * (later == 0.0)
    cnt = jnp.maximum(oh.sum(axis=3), 1.0)
    A = w / cnt[:, :, :, None, :]
    A_all = A.transpose(0, 4, 1, 2, 3).reshape(SEQ * NB * R, BN)

    loc0 = jnp.broadcast_to(current_location[None], (K, BN, 2))
    prev = jnp.concatenate([loc0[:, None], y_path[:, :-1]], axis=1)
    vel = (y_path - prev) * HZ
    yfv = vel.reshape(-1, 2) @ Wv + bv
    yfv_all = jnp.swapaxes(yfv.reshape(K, SEQ, BN, 16), 0, 1) \
        .reshape(SEQ * R, 16)

    bscf_x = jnp.concatenate(
        [jnp.zeros((48,), jnp.float32), bscf])
    Wr, Wz, Wn = Wih[:, 0:48], Wih[:, 48:96], Wih[:, 96:144]
    Ur, Uz, Un = Whh[:, 0:48], Whh[:, 48:96], Whh[:, 96:144]
    bir = (bih[0:48] + bscf_x @ Wr).reshape(1, 48)
    biz = (bih[48:96] + bscf_x @ Wz).reshape(1, 48)
    bin_ = (bih[96:144] + bscf_x @ Wn).reshape(1, 48)
    bhr = bhh[0:48].reshape(1, 48)
    bhz = bhh[48:96].reshape(1, 48)
    bhn = bhh[96:144].reshape(1, 48)

    smem = pl.BlockSpec(memory_space=pltpu.SMEM)

    out_dy, out_sc = pl.pallas_call(
        _body,
        out_shape=[
            jax.ShapeDtypeStruct((R, 80), jnp.float32),
            jax.ShapeDtypeStruct((R, 1), jnp.float32),
        ],
        in_specs=[pl.BlockSpec(memory_space=pltpu.VMEM),
                  pl.BlockSpec(memory_space=pltpu.VMEM),
                  smem,
                  pl.BlockSpec(memory_space=pltpu.VMEM),
                  pl.BlockSpec(memory_space=pltpu.VMEM),
                  ] + [pl.BlockSpec(memory_space=pltpu.VMEM)] * 17,
    )(hx, fm_flat, lin_all, A_all, yfv_all, Wscf,
      Wr, Wz, Wn, Ur, Uz, Un, bir, biz, bin_, bhr, bhz, bhn,
      Wdy, bdy.reshape(1, 80), Wsc, bsc.reshape(1, 1))

    deltaY = out_dy.reshape(K, BN, 2, SEQ).transpose(0, 3, 1, 2)
    score = out_sc.reshape(K, BN, 1)
    return (deltaY, score)
